# baseline (speedup 1.0000x reference)
import functools

import jax
import jax.numpy as jnp
from jax import lax
from jax.experimental import pallas as pl
from jax.experimental.pallas import tpu as pltpu
from jax.experimental.pallas import tpu_sc as plsc

N = 10000
E = 320000
NPAD = 10240
NC, NS, L = 2, 16, 16
RPT = NPAD // NS
NW = NC * NS
CB = 1280
NBM = 8
RING = 2
EPW = NBM * CB
EPAD = EPW * NW
BNS = 0.9999950000374997


def _sc_agg(rows3, cols3, ew3, tab, zer, NF):
    ISL = 4
    scratch = (
        [pltpu.VMEM((CB,), jnp.int32) for _ in range(2 * ISL)]
        + [pltpu.VMEM((CB,), jnp.float32) for _ in range(ISL)]
        + [pltpu.VMEM((CB,), jnp.float32) for _ in range(2 * RING * NF)]
        + [pltpu.VMEM_SHARED((NPAD,), jnp.float32) for _ in range(2 * NF)]
        + [pltpu.SemaphoreType.DMA] * (ISL + 2 * RING)
    )

    @functools.partial(
        pl.kernel,
        out_type=jax.ShapeDtypeStruct((NC, 1, NF * NPAD), jnp.float32),
        mesh=plsc.VectorSubcoreMesh(core_axis_name="c", subcore_axis_name="s"),
        scratch_types=scratch,
    )
    def k(rows_h, cols_h, ew_h, tab_h, zer_h, out_h, *refs):
        it = iter(refs)
        ridx = [next(it) for _ in range(ISL)]
        cidx = [next(it) for _ in range(ISL)]
        ewb = [next(it) for _ in range(ISL)]
        gval = [[next(it) for _ in range(NF)] for _ in range(RING)]
        mval = [[next(it) for _ in range(NF)] for _ in range(RING)]
        tabs = [next(it) for _ in range(NF)]
        accs = [next(it) for _ in range(NF)]
        isems = [next(it) for _ in range(ISL)]
        gsems = [next(it) for _ in range(RING)]
        ssems = [next(it) for _ in range(RING)]
        c = lax.axis_index("c")
        s = lax.axis_index("s")
        base = (s * NC + c) * EPW

        def fire_stage(j, sl):
            off = base + j * CB
            pltpu.async_copy(rows_h.at[pl.ds(off, CB)], ridx[sl], isems[sl])
            pltpu.async_copy(cols_h.at[pl.ds(off, CB)], cidx[sl], isems[sl])
            pltpu.async_copy(ew_h.at[pl.ds(off, CB)], ewb[sl], isems[sl])

        def wait_stage(sl):
            pltpu.make_async_copy(rows_h.at[pl.ds(0, CB)], ridx[sl],
                                  isems[sl]).wait()
            pltpu.make_async_copy(rows_h.at[pl.ds(0, CB)], cidx[sl],
                                  isems[sl]).wait()
            pltpu.make_async_copy(ew_h.at[pl.ds(0, CB)], ewb[sl],
                                  isems[sl]).wait()

        def fire_gathers(sl, cc):
            for f in range(NF):
                pltpu.async_copy(tabs[f].at[ridx[sl]], gval[cc][f], gsems[cc])

        def drain(sem, bufs):
            for f in range(NF):
                pltpu.make_async_copy(zer_h.at[pl.ds(0, CB)], bufs[f],
                                      sem).wait()

        for f in range(NF):
            @pl.when(s == f)
            def _(f=f):
                pltpu.sync_copy(tab_h.at[pl.ds(f * NPAD, NPAD)], tabs[f])
                pltpu.sync_copy(zer_h.at[pl.ds(f * NPAD, NPAD)], accs[f])

        for sl in range(ISL):
            fire_stage(sl, sl)
        plsc.subcore_barrier()
        wait_stage(0)
        fire_gathers(0, 0)

        def outer(jo, carry):
            for u in range(4):
                j = jo * 4 + u
                cc = u % 2
                nx = (u + 1) % 2
                isl = u
                @pl.when(j + 1 < NBM)
                def _():
                    wait_stage((u + 1) % ISL)
                    fire_gathers((u + 1) % ISL, nx)

                drain(gsems[cc], gval[cc])

                @pl.when(j >= 2)
                def _():
                    drain(ssems[cc], mval[cc])
                    @pl.when((j + 2 < NBM) & (j + 2 >= ISL))
                    def _():
                        fire_stage(j + 2, (u + 2) % ISL)

                def scale(k16, _):
                    sl = pl.ds(k16 * L, L)
                    ew16 = ewb[isl][sl]
                    for f in range(NF):
                        mval[cc][f][sl] = gval[cc][f][sl] * ew16
                    return 0

                lax.fori_loop(0, CB // L, scale, 0)

                for f in range(NF):
                    pltpu.async_copy(mval[cc][f], accs[f].at[cidx[isl]],
                                     ssems[cc], add=True)
            return carry

        lax.fori_loop(0, NBM // 4, outer, 0)
        for cc in range(RING):
            drain(ssems[cc], mval[cc])
        plsc.subcore_barrier()

        for f in range(NF):
            @pl.when(s == f)
            def _(f=f):
                pltpu.sync_copy(accs[f], out_h.at[c, 0, pl.ds(f * NPAD, NPAD)])

    out = k(rows3, cols3, ew3, tab, zer)
    return out.reshape(NC, NF, NPAD)


def _sc_agg_row(rows3, cols3, ew3, tab, zer):
    NF = 16
    ISL = 4
    scratch = (
        [pltpu.VMEM((CB,), jnp.int32) for _ in range(2 * ISL)]
        + [pltpu.VMEM((CB,), jnp.float32) for _ in range(ISL)]
        + [pltpu.VMEM((CB, NF), jnp.float32) for _ in range(2 * RING)]
        + [pltpu.VMEM_SHARED((NPAD, NF), jnp.float32) for _ in range(2)]
        + [pltpu.SemaphoreType.DMA] * (ISL + 2 * RING)
    )

    @functools.partial(
        pl.kernel,
        out_type=jax.ShapeDtypeStruct((NC, NPAD, NF), jnp.float32),
        mesh=plsc.VectorSubcoreMesh(core_axis_name="c", subcore_axis_name="s"),
        scratch_types=scratch,
        compiler_params=pltpu.CompilerParams(use_tc_tiling_on_sc=False),
    )
    def k(rows_h, cols_h, ew_h, tab_h, zer_h, out_h, *refs):
        it = iter(refs)
        ridx = [next(it) for _ in range(ISL)]
        cidx = [next(it) for _ in range(ISL)]
        ewb = [next(it) for _ in range(ISL)]
        gval = [next(it) for _ in range(RING)]
        mval = [next(it) for _ in range(RING)]
        tabs = next(it)
        accs = next(it)
        isems = [next(it) for _ in range(ISL)]
        gsems = [next(it) for _ in range(RING)]
        ssems = [next(it) for _ in range(RING)]
        c = lax.axis_index("c")
        s = lax.axis_index("s")
        base = (s * NC + c) * EPW

        def fire_stage(j, sl):
            off = base + j * CB
            pltpu.async_copy(rows_h.at[pl.ds(off, CB)], ridx[sl], isems[sl])
            pltpu.async_copy(cols_h.at[pl.ds(off, CB)], cidx[sl], isems[sl])
            pltpu.async_copy(ew_h.at[pl.ds(off, CB)], ewb[sl], isems[sl])

        def wait_stage(sl):
            pltpu.make_async_copy(rows_h.at[pl.ds(0, CB)], ridx[sl],
                                  isems[sl]).wait()
            pltpu.make_async_copy(rows_h.at[pl.ds(0, CB)], cidx[sl],
                                  isems[sl]).wait()
            pltpu.make_async_copy(ew_h.at[pl.ds(0, CB)], ewb[sl],
                                  isems[sl]).wait()

        r0 = s * RPT
        pltpu.sync_copy(tab_h.at[pl.ds(r0, RPT), :],
                        tabs.at[pl.ds(r0, RPT), :])
        pltpu.sync_copy(zer_h.at[pl.ds(r0, RPT), :],
                        accs.at[pl.ds(r0, RPT), :])
        for sl in range(ISL):
            fire_stage(sl, sl)
        plsc.subcore_barrier()
        wait_stage(0)
        pltpu.async_copy(tabs.at[ridx[0]], gval[0], gsems[0])

        def outer(jo, carry):
            for u in range(4):
                j = jo * 4 + u
                cc = u % 2
                nx = (u + 1) % 2

                @pl.when(j + 1 < NBM)
                def _():
                    wait_stage((u + 1) % ISL)
                    pltpu.async_copy(tabs.at[ridx[(u + 1) % ISL]], gval[nx],
                                     gsems[nx])

                pltpu.make_async_copy(zer_h.at[pl.ds(0, CB), :], gval[cc],
                                      gsems[cc]).wait()

                @pl.when(j >= 2)
                def _():
                    pltpu.make_async_copy(zer_h.at[pl.ds(0, CB), :], mval[cc],
                                          ssems[cc]).wait()

                    @pl.when(j + 2 < NBM)
                    def _():
                        fire_stage(j + 2, (u + 2) % ISL)

                def scale(k16, _):
                    b = k16 * L
                    ew16 = ewb[u][pl.ds(b, L)]
                    for kk in range(L):
                        mval[cc][b + kk, :] = gval[cc][b + kk, :] * ew16[kk]
                    return 0

                lax.fori_loop(0, CB // L, scale, 0)

                pltpu.async_copy(mval[cc], accs.at[cidx[u]], ssems[cc],
                                 add=True)
            return carry

        lax.fori_loop(0, NBM // 4, outer, 0)
        for cc in range(RING):
            pltpu.make_async_copy(zer_h.at[pl.ds(0, CB), :], mval[cc],
                                  ssems[cc]).wait()
        plsc.subcore_barrier()
        pltpu.sync_copy(accs.at[pl.ds(r0, RPT), :],
                        out_h.at[c, pl.ds(r0, RPT), :])

    return k(rows3, cols3, ew3, tab, zer)


def _tc_h1(x, W1):
    def body(x_ref, w_ref, h_ref):
        h = jnp.dot(x_ref[...], w_ref[...],
                    preferred_element_type=jnp.float32)
        h_ref[...] = jnp.concatenate(
            [h, jnp.zeros((NPAD - N, 16), jnp.float32)], axis=0)

    return pl.pallas_call(
        body,
        out_shape=jax.ShapeDtypeStruct((NPAD, 16), jnp.float32),
    )(x, W1)


def _tc_scale1(degp, h1):
    def body(dp_ref, h_ref, g_ref, dinv_ref):
        dp = dp_ref[...]
        deg = 1.0 + dp[0, 0, :N] + dp[1, 0, :N]
        dinv = lax.rsqrt(deg)[:, None]
        g = dinv * h_ref[:N, :]
        g_ref[...] = jnp.concatenate(
            [g, jnp.zeros((NPAD - N, 16), jnp.float32)], axis=0)
        dinv_ref[...] = dinv

    return pl.pallas_call(
        body,
        out_shape=(jax.ShapeDtypeStruct((NPAD, 16), jnp.float32),
                   jax.ShapeDtypeStruct((N, 1), jnp.float32)),
    )(degp, h1)


def _tc_mid(agg, gv, dinv, bias, gam, bet, Wn, d_in, d_out):
    def body(a_ref, g_ref, di_ref, b_ref, ga_ref, be_ref, w_ref, o_ref):
        a = (a_ref[0] + a_ref[1] + g_ref[...])[:N, :d_in]
        dinv = di_ref[...]
        out = dinv * a + b_ref[...]
        bn = ga_ref[...] * (out * BNS) + be_ref[...]
        r = jnp.maximum(bn, 0.0)
        h = jnp.dot(r, w_ref[...], preferred_element_type=jnp.float32)
        g_next = h * dinv
        if d_out > 1:
            g_next = jnp.concatenate(
                [g_next, jnp.zeros((N, 16 - d_out), jnp.float32)], axis=1)
            g_next = jnp.concatenate(
                [g_next, jnp.zeros((NPAD - N, 16), jnp.float32)], axis=0)
        o_ref[...] = g_next

    out_shape = (jax.ShapeDtypeStruct((NPAD, 16), jnp.float32) if d_out > 1
                 else jax.ShapeDtypeStruct((N, 1), jnp.float32))
    return pl.pallas_call(
        body,
        out_shape=out_shape,
    )(agg, gv, dinv, bias, gam, bet, Wn)


def _tc_post(aggp, gv, dinv, bias, gam, bet, Wl, bl):
    def body(a_ref, g_ref, di_ref, b_ref, ga_ref, be_ref, wl_ref, bl_ref,
             o_ref):
        a = (a_ref[0, 0, :N] + a_ref[1, 0, :N])[:, None] + g_ref[...]
        out = di_ref[...] * a + b_ref[...]
        bn = ga_ref[...] * (out * BNS) + be_ref[...]
        z = bn * wl_ref[...] + bl_ref[...]
        o_ref[...] = jax.nn.sigmoid(z)

    return pl.pallas_call(
        body,
        out_shape=jax.ShapeDtypeStruct((N, 1), jnp.float32),
    )(aggp, gv, dinv, bias, gam, bet, Wl, bl)


def kernel(x, edge_index, edge_attr, W1, b1, g1, be1, W2, b2, g2, be2,
           W3, b3, g3, be3, Wl, bl):
    row = edge_index[0].astype(jnp.int32)
    col = edge_index[1].astype(jnp.int32)
    ew = edge_attr.astype(jnp.float32)
    pad = EPAD - E
    rows3 = jnp.pad(row, (0, pad))
    cols3 = jnp.pad(col, (0, pad))
    ew3 = jnp.pad(ew, (0, pad))
    zer16 = jnp.zeros((16 * NPAD,), jnp.float32)
    ones_t = jnp.ones((NPAD,), jnp.float32)

    h1 = _tc_h1(x, W1)
    degp = _sc_agg(rows3, cols3, ew3, ones_t, zer16[:NPAD], 1)

    g1n, dinv = _tc_scale1(degp, h1)
    zer16n = jnp.zeros((NPAD, 16), jnp.float32)
    ag1n = _sc_agg_row(rows3, cols3, ew3, g1n, zer16n)
    g2n = _tc_mid(ag1n, g1n, dinv, b1.reshape(1, 16), g1.reshape(1, 16),
                  be1.reshape(1, 16), W2, 16, 4)

    ag2n = _sc_agg_row(rows3, cols3, ew3, g2n, zer16n)
    g3n = _tc_mid(ag2n, g2n, dinv, b2.reshape(1, 4), g2.reshape(1, 4),
                  be2.reshape(1, 4), W3, 4, 1)

    ag3 = _sc_agg(rows3, cols3, ew3, g3n.reshape(-1), zer16[:NPAD], 1)
    y = _tc_post(ag3, g3n, dinv, b3.reshape(1, 1), g3.reshape(1, 1),
                 be3.reshape(1, 1), Wl, bl.reshape(1, 1))
    return y


# speedup vs baseline: 1.0226x; 1.0226x over previous
import functools

import jax
import jax.numpy as jnp
from jax import lax
from jax.experimental import pallas as pl
from jax.experimental.pallas import tpu as pltpu
from jax.experimental.pallas import tpu_sc as plsc

N = 10000
E = 320000
NPAD = 10240
NC, NS, L = 2, 16, 16
RPT = NPAD // NS
NW = NC * NS
CB = 512
NBM = 20
RING = 2
EPW = NBM * CB
EPAD = EPW * NW
BNS = 0.9999950000374997


def _sc_agg(rows3, cols3, ew3, tab, zer, NF):
    ISL = 4
    scratch = (
        [pltpu.VMEM((CB,), jnp.int32) for _ in range(2 * ISL)]
        + [pltpu.VMEM((CB,), jnp.float32) for _ in range(ISL)]
        + [pltpu.VMEM((CB,), jnp.float32) for _ in range(2 * RING * NF)]
        + [pltpu.VMEM_SHARED((NPAD,), jnp.float32) for _ in range(2 * NF)]
        + [pltpu.SemaphoreType.DMA] * (ISL + 2 * RING)
    )

    @functools.partial(
        pl.kernel,
        out_type=jax.ShapeDtypeStruct((NC, 1, NF * NPAD), jnp.float32),
        mesh=plsc.VectorSubcoreMesh(core_axis_name="c", subcore_axis_name="s"),
        scratch_types=scratch,
    )
    def k(rows_h, cols_h, ew_h, tab_h, zer_h, out_h, *refs):
        it = iter(refs)
        ridx = [next(it) for _ in range(ISL)]
        cidx = [next(it) for _ in range(ISL)]
        ewb = [next(it) for _ in range(ISL)]
        gval = [[next(it) for _ in range(NF)] for _ in range(RING)]
        mval = [[next(it) for _ in range(NF)] for _ in range(RING)]
        tabs = [next(it) for _ in range(NF)]
        accs = [next(it) for _ in range(NF)]
        isems = [next(it) for _ in range(ISL)]
        gsems = [next(it) for _ in range(RING)]
        ssems = [next(it) for _ in range(RING)]
        c = lax.axis_index("c")
        s = lax.axis_index("s")
        base = (s * NC + c) * EPW

        def fire_stage(j, sl):
            off = base + j * CB
            pltpu.async_copy(rows_h.at[pl.ds(off, CB)], ridx[sl], isems[sl])
            pltpu.async_copy(cols_h.at[pl.ds(off, CB)], cidx[sl], isems[sl])
            pltpu.async_copy(ew_h.at[pl.ds(off, CB)], ewb[sl], isems[sl])

        def wait_stage(sl):
            pltpu.make_async_copy(rows_h.at[pl.ds(0, CB)], ridx[sl],
                                  isems[sl]).wait()
            pltpu.make_async_copy(rows_h.at[pl.ds(0, CB)], cidx[sl],
                                  isems[sl]).wait()
            pltpu.make_async_copy(ew_h.at[pl.ds(0, CB)], ewb[sl],
                                  isems[sl]).wait()

        def fire_gathers(sl, cc):
            for f in range(NF):
                pltpu.async_copy(tabs[f].at[ridx[sl]], gval[cc][f], gsems[cc])

        def drain(sem, bufs):
            for f in range(NF):
                pltpu.make_async_copy(zer_h.at[pl.ds(0, CB)], bufs[f],
                                      sem).wait()

        for f in range(NF):
            @pl.when(s == f)
            def _(f=f):
                pltpu.sync_copy(tab_h.at[pl.ds(f * NPAD, NPAD)], tabs[f])
                pltpu.sync_copy(zer_h.at[pl.ds(f * NPAD, NPAD)], accs[f])

        for sl in range(ISL):
            fire_stage(sl, sl)
        plsc.subcore_barrier()
        wait_stage(0)
        fire_gathers(0, 0)

        def outer(jo, carry):
            for u in range(4):
                j = jo * 4 + u
                cc = u % 2
                nx = (u + 1) % 2
                isl = u
                @pl.when(j + 1 < NBM)
                def _():
                    wait_stage((u + 1) % ISL)
                    fire_gathers((u + 1) % ISL, nx)

                drain(gsems[cc], gval[cc])

                @pl.when(j >= 2)
                def _():
                    drain(ssems[cc], mval[cc])
                    @pl.when((j + 2 < NBM) & (j + 2 >= ISL))
                    def _():
                        fire_stage(j + 2, (u + 2) % ISL)

                def scale(k16, _):
                    sl = pl.ds(k16 * L, L)
                    ew16 = ewb[isl][sl]
                    for f in range(NF):
                        mval[cc][f][sl] = gval[cc][f][sl] * ew16
                    return 0

                lax.fori_loop(0, CB // L, scale, 0)

                for f in range(NF):
                    pltpu.async_copy(mval[cc][f], accs[f].at[cidx[isl]],
                                     ssems[cc], add=True)
            return carry

        lax.fori_loop(0, NBM // 4, outer, 0)
        for cc in range(RING):
            drain(ssems[cc], mval[cc])
        plsc.subcore_barrier()

        for f in range(NF):
            @pl.when(s == f)
            def _(f=f):
                pltpu.sync_copy(accs[f], out_h.at[c, 0, pl.ds(f * NPAD, NPAD)])

    out = k(rows3, cols3, ew3, tab, zer)
    return out.reshape(NC, NF, NPAD)


def _sc_agg_row(rows3, cols3, ew3, tab, zer):
    NF = 16
    ISL = 4
    scratch = (
        [pltpu.VMEM((CB,), jnp.int32) for _ in range(2 * ISL)]
        + [pltpu.VMEM((CB,), jnp.float32) for _ in range(ISL)]
        + [pltpu.VMEM((CB, NF), jnp.float32) for _ in range(2 * RING)]
        + [pltpu.VMEM_SHARED((NPAD, NF), jnp.float32) for _ in range(2)]
        + [pltpu.SemaphoreType.DMA] * (ISL + 2 * RING)
    )

    @functools.partial(
        pl.kernel,
        out_type=jax.ShapeDtypeStruct((NC, NPAD, NF), jnp.float32),
        mesh=plsc.VectorSubcoreMesh(core_axis_name="c", subcore_axis_name="s"),
        scratch_types=scratch,
        compiler_params=pltpu.CompilerParams(use_tc_tiling_on_sc=False),
    )
    def k(rows_h, cols_h, ew_h, tab_h, zer_h, out_h, *refs):
        it = iter(refs)
        ridx = [next(it) for _ in range(ISL)]
        cidx = [next(it) for _ in range(ISL)]
        ewb = [next(it) for _ in range(ISL)]
        gval = [next(it) for _ in range(RING)]
        mval = [next(it) for _ in range(RING)]
        tabs = next(it)
        accs = next(it)
        isems = [next(it) for _ in range(ISL)]
        gsems = [next(it) for _ in range(RING)]
        ssems = [next(it) for _ in range(RING)]
        c = lax.axis_index("c")
        s = lax.axis_index("s")
        base = (s * NC + c) * EPW

        def fire_stage(j, sl):
            off = base + j * CB
            pltpu.async_copy(rows_h.at[pl.ds(off, CB)], ridx[sl], isems[sl])
            pltpu.async_copy(cols_h.at[pl.ds(off, CB)], cidx[sl], isems[sl])
            pltpu.async_copy(ew_h.at[pl.ds(off, CB)], ewb[sl], isems[sl])

        def wait_stage(sl):
            pltpu.make_async_copy(rows_h.at[pl.ds(0, CB)], ridx[sl],
                                  isems[sl]).wait()
            pltpu.make_async_copy(rows_h.at[pl.ds(0, CB)], cidx[sl],
                                  isems[sl]).wait()
            pltpu.make_async_copy(ew_h.at[pl.ds(0, CB)], ewb[sl],
                                  isems[sl]).wait()

        r0 = s * RPT
        pltpu.sync_copy(tab_h.at[pl.ds(r0, RPT), :],
                        tabs.at[pl.ds(r0, RPT), :])
        pltpu.sync_copy(zer_h.at[pl.ds(r0, RPT), :],
                        accs.at[pl.ds(r0, RPT), :])
        for sl in range(ISL):
            fire_stage(sl, sl)
        plsc.subcore_barrier()
        wait_stage(0)
        pltpu.async_copy(tabs.at[ridx[0]], gval[0], gsems[0])

        def outer(jo, carry):
            for u in range(4):
                j = jo * 4 + u
                cc = u % 2
                nx = (u + 1) % 2

                @pl.when(j + 1 < NBM)
                def _():
                    wait_stage((u + 1) % ISL)
                    pltpu.async_copy(tabs.at[ridx[(u + 1) % ISL]], gval[nx],
                                     gsems[nx])

                pltpu.make_async_copy(zer_h.at[pl.ds(0, CB), :], gval[cc],
                                      gsems[cc]).wait()

                @pl.when(j >= 2)
                def _():
                    pltpu.make_async_copy(zer_h.at[pl.ds(0, CB), :], mval[cc],
                                          ssems[cc]).wait()

                    @pl.when(j + 2 < NBM)
                    def _():
                        fire_stage(j + 2, (u + 2) % ISL)

                def scale(k16, _):
                    b = k16 * L
                    ew16 = ewb[u][pl.ds(b, L)]
                    for kk in range(L):
                        mval[cc][b + kk, :] = gval[cc][b + kk, :] * ew16[kk]
                    return 0

                lax.fori_loop(0, CB // L, scale, 0)

                pltpu.async_copy(mval[cc], accs.at[cidx[u]], ssems[cc],
                                 add=True)
            return carry

        lax.fori_loop(0, NBM // 4, outer, 0)
        for cc in range(RING):
            pltpu.make_async_copy(zer_h.at[pl.ds(0, CB), :], mval[cc],
                                  ssems[cc]).wait()
        plsc.subcore_barrier()
        pltpu.sync_copy(accs.at[pl.ds(r0, RPT), :],
                        out_h.at[c, pl.ds(r0, RPT), :])

    return k(rows3, cols3, ew3, tab, zer)


def _tc_h1(x, W1):
    def body(x_ref, w_ref, h_ref):
        h = jnp.dot(x_ref[...], w_ref[...],
                    preferred_element_type=jnp.float32)
        h_ref[...] = jnp.concatenate(
            [h, jnp.zeros((NPAD - N, 16), jnp.float32)], axis=0)

    return pl.pallas_call(
        body,
        out_shape=jax.ShapeDtypeStruct((NPAD, 16), jnp.float32),
    )(x, W1)


def _tc_scale1(degp, h1):
    def body(dp_ref, h_ref, g_ref, dinv_ref):
        dp = dp_ref[...]
        deg = 1.0 + dp[0, 0, :N] + dp[1, 0, :N]
        dinv = lax.rsqrt(deg)[:, None]
        g = dinv * h_ref[:N, :]
        g_ref[...] = jnp.concatenate(
            [g, jnp.zeros((NPAD - N, 16), jnp.float32)], axis=0)
        dinv_ref[...] = dinv

    return pl.pallas_call(
        body,
        out_shape=(jax.ShapeDtypeStruct((NPAD, 16), jnp.float32),
                   jax.ShapeDtypeStruct((N, 1), jnp.float32)),
    )(degp, h1)


def _tc_mid(agg, gv, dinv, bias, gam, bet, Wn, d_in, d_out):
    def body(a_ref, g_ref, di_ref, b_ref, ga_ref, be_ref, w_ref, o_ref):
        a = (a_ref[0] + a_ref[1] + g_ref[...])[:N, :d_in]
        dinv = di_ref[...]
        out = dinv * a + b_ref[...]
        bn = ga_ref[...] * (out * BNS) + be_ref[...]
        r = jnp.maximum(bn, 0.0)
        h = jnp.dot(r, w_ref[...], preferred_element_type=jnp.float32)
        g_next = h * dinv
        if d_out > 1:
            g_next = jnp.concatenate(
                [g_next, jnp.zeros((N, 16 - d_out), jnp.float32)], axis=1)
            g_next = jnp.concatenate(
                [g_next, jnp.zeros((NPAD - N, 16), jnp.float32)], axis=0)
        o_ref[...] = g_next

    out_shape = (jax.ShapeDtypeStruct((NPAD, 16), jnp.float32) if d_out > 1
                 else jax.ShapeDtypeStruct((N, 1), jnp.float32))
    return pl.pallas_call(
        body,
        out_shape=out_shape,
    )(agg, gv, dinv, bias, gam, bet, Wn)


def _tc_post(aggp, gv, dinv, bias, gam, bet, Wl, bl):
    def body(a_ref, g_ref, di_ref, b_ref, ga_ref, be_ref, wl_ref, bl_ref,
             o_ref):
        a = (a_ref[0, 0, :N] + a_ref[1, 0, :N])[:, None] + g_ref[...]
        out = di_ref[...] * a + b_ref[...]
        bn = ga_ref[...] * (out * BNS) + be_ref[...]
        z = bn * wl_ref[...] + bl_ref[...]
        o_ref[...] = jax.nn.sigmoid(z)

    return pl.pallas_call(
        body,
        out_shape=jax.ShapeDtypeStruct((N, 1), jnp.float32),
    )(aggp, gv, dinv, bias, gam, bet, Wl, bl)


def kernel(x, edge_index, edge_attr, W1, b1, g1, be1, W2, b2, g2, be2,
           W3, b3, g3, be3, Wl, bl):
    row = edge_index[0].astype(jnp.int32)
    col = edge_index[1].astype(jnp.int32)
    ew = edge_attr.astype(jnp.float32)
    pad = EPAD - E
    rows3 = jnp.pad(row, (0, pad))
    cols3 = jnp.pad(col, (0, pad))
    ew3 = jnp.pad(ew, (0, pad))
    zer16 = jnp.zeros((16 * NPAD,), jnp.float32)
    ones_t = jnp.ones((NPAD,), jnp.float32)

    h1 = _tc_h1(x, W1)
    degp = _sc_agg(rows3, cols3, ew3, ones_t, zer16[:NPAD], 1)

    g1n, dinv = _tc_scale1(degp, h1)
    zer16n = jnp.zeros((NPAD, 16), jnp.float32)
    ag1n = _sc_agg_row(rows3, cols3, ew3, g1n, zer16n)
    g2n = _tc_mid(ag1n, g1n, dinv, b1.reshape(1, 16), g1.reshape(1, 16),
                  be1.reshape(1, 16), W2, 16, 4)

    ag2n = _sc_agg_row(rows3, cols3, ew3, g2n, zer16n)
    g3n = _tc_mid(ag2n, g2n, dinv, b2.reshape(1, 4), g2.reshape(1, 4),
                  be2.reshape(1, 4), W3, 4, 1)

    ag3 = _sc_agg(rows3, cols3, ew3, g3n.reshape(-1), zer16[:NPAD], 1)
    y = _tc_post(ag3, g3n, dinv, b3.reshape(1, 1), g3.reshape(1, 1),
                 be3.reshape(1, 1), Wl, bl.reshape(1, 1))
    return y


# speedup vs baseline: 1.0504x; 1.0272x over previous
import functools

import jax
import jax.numpy as jnp
from jax import lax
from jax.experimental import pallas as pl
from jax.experimental.pallas import tpu as pltpu
from jax.experimental.pallas import tpu_sc as plsc

N = 10000
E = 320000
NPAD = 10240
NC, NS, L = 2, 16, 16
RPT = NPAD // NS
NW = NC * NS
CB = 512
NBM = 20
RING = 2
EPW = NBM * CB
EPAD = EPW * NW
BNS = 0.9999950000374997


def _sc_agg(rows3, cols3, ew3, tab, zer, NF, weights_only=False):
    ISL = 4
    scratch = (
        [pltpu.VMEM((CB,), jnp.int32) for _ in range(2 * ISL)]
        + [pltpu.VMEM((CB,), jnp.float32) for _ in range(ISL)]
        + [pltpu.VMEM((CB,), jnp.float32) for _ in range(2 * RING * NF)]
        + [pltpu.VMEM_SHARED((NPAD,), jnp.float32) for _ in range(2 * NF)]
        + [pltpu.SemaphoreType.DMA] * (ISL + 2 * RING)
    )

    @functools.partial(
        pl.kernel,
        out_type=jax.ShapeDtypeStruct((NC, 1, NF * NPAD), jnp.float32),
        mesh=plsc.VectorSubcoreMesh(core_axis_name="c", subcore_axis_name="s"),
        scratch_types=scratch,
    )
    def k(rows_h, cols_h, ew_h, tab_h, zer_h, out_h, *refs):
        it = iter(refs)
        ridx = [next(it) for _ in range(ISL)]
        cidx = [next(it) for _ in range(ISL)]
        ewb = [next(it) for _ in range(ISL)]
        gval = [[next(it) for _ in range(NF)] for _ in range(RING)]
        mval = [[next(it) for _ in range(NF)] for _ in range(RING)]
        tabs = [next(it) for _ in range(NF)]
        accs = [next(it) for _ in range(NF)]
        isems = [next(it) for _ in range(ISL)]
        gsems = [next(it) for _ in range(RING)]
        ssems = [next(it) for _ in range(RING)]
        c = lax.axis_index("c")
        s = lax.axis_index("s")
        base = (s * NC + c) * EPW

        def fire_stage(j, sl):
            off = base + j * CB
            if not weights_only:
                pltpu.async_copy(rows_h.at[pl.ds(off, CB)], ridx[sl],
                                 isems[sl])
            pltpu.async_copy(cols_h.at[pl.ds(off, CB)], cidx[sl], isems[sl])
            pltpu.async_copy(ew_h.at[pl.ds(off, CB)], ewb[sl], isems[sl])

        def wait_stage(sl):
            if not weights_only:
                pltpu.make_async_copy(rows_h.at[pl.ds(0, CB)], ridx[sl],
                                      isems[sl]).wait()
            pltpu.make_async_copy(rows_h.at[pl.ds(0, CB)], cidx[sl],
                                  isems[sl]).wait()
            pltpu.make_async_copy(ew_h.at[pl.ds(0, CB)], ewb[sl],
                                  isems[sl]).wait()

        def fire_gathers(sl, cc):
            for f in range(NF):
                pltpu.async_copy(tabs[f].at[ridx[sl]], gval[cc][f], gsems[cc])

        def drain(sem, bufs):
            for f in range(NF):
                pltpu.make_async_copy(zer_h.at[pl.ds(0, CB)], bufs[f],
                                      sem).wait()

        for f in range(NF):
            @pl.when(s == f)
            def _(f=f):
                if not weights_only:
                    pltpu.sync_copy(tab_h.at[pl.ds(f * NPAD, NPAD)], tabs[f])
                pltpu.sync_copy(zer_h.at[pl.ds(f * NPAD, NPAD)], accs[f])

        for sl in range(ISL):
            fire_stage(sl, sl)
        plsc.subcore_barrier()
        if not weights_only:
            wait_stage(0)
            fire_gathers(0, 0)

        def outer(jo, carry):
            for u in range(4):
                j = jo * 4 + u
                cc = u % 2
                nx = (u + 1) % 2
                isl = u
                if not weights_only:
                    @pl.when(j + 1 < NBM)
                    def _():
                        wait_stage((u + 1) % ISL)
                        fire_gathers((u + 1) % ISL, nx)

                    drain(gsems[cc], gval[cc])
                else:
                    wait_stage(u)

                @pl.when(j >= 2)
                def _():
                    drain(ssems[cc], mval[cc])
                    @pl.when((j + 2 < NBM) & (j + 2 >= ISL))
                    def _():
                        fire_stage(j + 2, (u + 2) % ISL)

                if weights_only:
                    pltpu.async_copy(ewb[u], accs[0].at[cidx[u]],
                                     ssems[cc], add=True)
                else:
                    def scale(k16, _):
                        sl = pl.ds(k16 * L, L)
                        ew16 = ewb[isl][sl]
                        for f in range(NF):
                            mval[cc][f][sl] = gval[cc][f][sl] * ew16
                        return 0

                    lax.fori_loop(0, CB // L, scale, 0)

                    for f in range(NF):
                        pltpu.async_copy(mval[cc][f], accs[f].at[cidx[isl]],
                                         ssems[cc], add=True)
            return carry

        lax.fori_loop(0, NBM // 4, outer, 0)
        for cc in range(RING):
            drain(ssems[cc], mval[cc])
        plsc.subcore_barrier()

        for f in range(NF):
            @pl.when(s == f)
            def _(f=f):
                pltpu.sync_copy(accs[f], out_h.at[c, 0, pl.ds(f * NPAD, NPAD)])

    out = k(rows3, cols3, ew3, tab, zer)
    return out.reshape(NC, NF, NPAD)


def _sc_agg_row(rows3, cols3, ew3, tab, zer):
    NF = 16
    ISL = 4
    scratch = (
        [pltpu.VMEM((CB,), jnp.int32) for _ in range(2 * ISL)]
        + [pltpu.VMEM((CB,), jnp.float32) for _ in range(ISL)]
        + [pltpu.VMEM((CB, NF), jnp.float32) for _ in range(2 * RING)]
        + [pltpu.VMEM_SHARED((NPAD, NF), jnp.float32) for _ in range(2)]
        + [pltpu.SemaphoreType.DMA] * (ISL + 2 * RING)
    )

    @functools.partial(
        pl.kernel,
        out_type=jax.ShapeDtypeStruct((NC, NPAD, NF), jnp.float32),
        mesh=plsc.VectorSubcoreMesh(core_axis_name="c", subcore_axis_name="s"),
        scratch_types=scratch,
        compiler_params=pltpu.CompilerParams(use_tc_tiling_on_sc=False),
    )
    def k(rows_h, cols_h, ew_h, tab_h, zer_h, out_h, *refs):
        it = iter(refs)
        ridx = [next(it) for _ in range(ISL)]
        cidx = [next(it) for _ in range(ISL)]
        ewb = [next(it) for _ in range(ISL)]
        gval = [next(it) for _ in range(RING)]
        mval = [next(it) for _ in range(RING)]
        tabs = next(it)
        accs = next(it)
        isems = [next(it) for _ in range(ISL)]
        gsems = [next(it) for _ in range(RING)]
        ssems = [next(it) for _ in range(RING)]
        c = lax.axis_index("c")
        s = lax.axis_index("s")
        base = (s * NC + c) * EPW

        def fire_stage(j, sl):
            off = base + j * CB
            pltpu.async_copy(rows_h.at[pl.ds(off, CB)], ridx[sl], isems[sl])
            pltpu.async_copy(cols_h.at[pl.ds(off, CB)], cidx[sl], isems[sl])
            pltpu.async_copy(ew_h.at[pl.ds(off, CB)], ewb[sl], isems[sl])

        def wait_stage(sl):
            pltpu.make_async_copy(rows_h.at[pl.ds(0, CB)], ridx[sl],
                                  isems[sl]).wait()
            pltpu.make_async_copy(rows_h.at[pl.ds(0, CB)], cidx[sl],
                                  isems[sl]).wait()
            pltpu.make_async_copy(ew_h.at[pl.ds(0, CB)], ewb[sl],
                                  isems[sl]).wait()

        r0 = s * RPT
        pltpu.sync_copy(tab_h.at[pl.ds(r0, RPT), :],
                        tabs.at[pl.ds(r0, RPT), :])
        pltpu.sync_copy(zer_h.at[pl.ds(r0, RPT), :],
                        accs.at[pl.ds(r0, RPT), :])
        for sl in range(ISL):
            fire_stage(sl, sl)
        plsc.subcore_barrier()
        wait_stage(0)
        pltpu.async_copy(tabs.at[ridx[0]], gval[0], gsems[0])

        def outer(jo, carry):
            for u in range(4):
                j = jo * 4 + u
                cc = u % 2
                nx = (u + 1) % 2

                @pl.when(j + 1 < NBM)
                def _():
                    wait_stage((u + 1) % ISL)
                    pltpu.async_copy(tabs.at[ridx[(u + 1) % ISL]], gval[nx],
                                     gsems[nx])

                pltpu.make_async_copy(zer_h.at[pl.ds(0, CB), :], gval[cc],
                                      gsems[cc]).wait()

                @pl.when(j >= 2)
                def _():
                    pltpu.make_async_copy(zer_h.at[pl.ds(0, CB), :], mval[cc],
                                          ssems[cc]).wait()

                    @pl.when(j + 2 < NBM)
                    def _():
                        fire_stage(j + 2, (u + 2) % ISL)

                def scale(k16, _):
                    b = k16 * L
                    ew16 = ewb[u][pl.ds(b, L)]
                    for kk in range(L):
                        mval[cc][b + kk, :] = gval[cc][b + kk, :] * ew16[kk]
                    return 0

                lax.fori_loop(0, CB // L, scale, 0)

                pltpu.async_copy(mval[cc], accs.at[cidx[u]], ssems[cc],
                                 add=True)
            return carry

        lax.fori_loop(0, NBM // 4, outer, 0)
        for cc in range(RING):
            pltpu.make_async_copy(zer_h.at[pl.ds(0, CB), :], mval[cc],
                                  ssems[cc]).wait()
        plsc.subcore_barrier()
        pltpu.sync_copy(accs.at[pl.ds(r0, RPT), :],
                        out_h.at[c, pl.ds(r0, RPT), :])

    return k(rows3, cols3, ew3, tab, zer)


def _tc_h1(x, W1):
    def body(x_ref, w_ref, h_ref):
        h = jnp.dot(x_ref[...], w_ref[...],
                    preferred_element_type=jnp.float32)
        h_ref[...] = jnp.concatenate(
            [h, jnp.zeros((NPAD - N, 16), jnp.float32)], axis=0)

    return pl.pallas_call(
        body,
        out_shape=jax.ShapeDtypeStruct((NPAD, 16), jnp.float32),
    )(x, W1)


def _tc_scale1(degp, h1):
    def body(dp_ref, h_ref, g_ref, dinv_ref):
        dp = dp_ref[...]
        deg = 1.0 + dp[0, 0, :N] + dp[1, 0, :N]
        dinv = lax.rsqrt(deg)[:, None]
        g = dinv * h_ref[:N, :]
        g_ref[...] = jnp.concatenate(
            [g, jnp.zeros((NPAD - N, 16), jnp.float32)], axis=0)
        dinv_ref[...] = dinv

    return pl.pallas_call(
        body,
        out_shape=(jax.ShapeDtypeStruct((NPAD, 16), jnp.float32),
                   jax.ShapeDtypeStruct((N, 1), jnp.float32)),
    )(degp, h1)


def _tc_mid(agg, gv, dinv, bias, gam, bet, Wn, d_in, d_out):
    def body(a_ref, g_ref, di_ref, b_ref, ga_ref, be_ref, w_ref, o_ref):
        a = (a_ref[0] + a_ref[1] + g_ref[...])[:N, :d_in]
        dinv = di_ref[...]
        out = dinv * a + b_ref[...]
        bn = ga_ref[...] * (out * BNS) + be_ref[...]
        r = jnp.maximum(bn, 0.0)
        h = jnp.dot(r, w_ref[...], preferred_element_type=jnp.float32)
        g_next = h * dinv
        if d_out > 1:
            g_next = jnp.concatenate(
                [g_next, jnp.zeros((N, 16 - d_out), jnp.float32)], axis=1)
            g_next = jnp.concatenate(
                [g_next, jnp.zeros((NPAD - N, 16), jnp.float32)], axis=0)
        o_ref[...] = g_next

    out_shape = (jax.ShapeDtypeStruct((NPAD, 16), jnp.float32) if d_out > 1
                 else jax.ShapeDtypeStruct((N, 1), jnp.float32))
    return pl.pallas_call(
        body,
        out_shape=out_shape,
    )(agg, gv, dinv, bias, gam, bet, Wn)


def _tc_post(aggp, gv, dinv, bias, gam, bet, Wl, bl):
    def body(a_ref, g_ref, di_ref, b_ref, ga_ref, be_ref, wl_ref, bl_ref,
             o_ref):
        a = (a_ref[0, 0, :N] + a_ref[1, 0, :N])[:, None] + g_ref[...]
        out = di_ref[...] * a + b_ref[...]
        bn = ga_ref[...] * (out * BNS) + be_ref[...]
        z = bn * wl_ref[...] + bl_ref[...]
        o_ref[...] = jax.nn.sigmoid(z)

    return pl.pallas_call(
        body,
        out_shape=jax.ShapeDtypeStruct((N, 1), jnp.float32),
    )(aggp, gv, dinv, bias, gam, bet, Wl, bl)


def kernel(x, edge_index, edge_attr, W1, b1, g1, be1, W2, b2, g2, be2,
           W3, b3, g3, be3, Wl, bl):
    row = edge_index[0].astype(jnp.int32)
    col = edge_index[1].astype(jnp.int32)
    ew = edge_attr.astype(jnp.float32)
    pad = EPAD - E
    rows3 = jnp.pad(row, (0, pad))
    cols3 = jnp.pad(col, (0, pad))
    ew3 = jnp.pad(ew, (0, pad))
    zer16 = jnp.zeros((16 * NPAD,), jnp.float32)
    ones_t = jnp.ones((NPAD,), jnp.float32)

    h1 = _tc_h1(x, W1)
    degp = _sc_agg(rows3, cols3, ew3, ones_t, zer16[:NPAD], 1,
                   weights_only=True)

    g1n, dinv = _tc_scale1(degp, h1)
    zer16n = jnp.zeros((NPAD, 16), jnp.float32)
    ag1n = _sc_agg_row(rows3, cols3, ew3, g1n, zer16n)
    g2n = _tc_mid(ag1n, g1n, dinv, b1.reshape(1, 16), g1.reshape(1, 16),
                  be1.reshape(1, 16), W2, 16, 4)

    ag2n = _sc_agg_row(rows3, cols3, ew3, g2n, zer16n)
    g3n = _tc_mid(ag2n, g2n, dinv, b2.reshape(1, 4), g2.reshape(1, 4),
                  be2.reshape(1, 4), W3, 4, 1)

    ag3 = _sc_agg(rows3, cols3, ew3, g3n.reshape(-1), zer16[:NPAD], 1)
    y = _tc_post(ag3, g3n, dinv, b3.reshape(1, 1), g3.reshape(1, 1),
                 be3.reshape(1, 1), Wl, bl.reshape(1, 1))
    return y
